# baseline (speedup 1.0000x reference)
import functools

import jax
import jax.numpy as jnp
from jax import lax
from jax.experimental import pallas as pl
from jax.experimental.pallas import tpu as pltpu
from jax.experimental.pallas import tpu_sc as plsc

D = 768
BATCH = 4
SEQ = 2048
LANES = 16
VECS = D // LANES

_info = plsc.get_sparse_core_info()
NC = _info.num_cores
NS = _info.num_subcores
NW = NC * NS
S_PER_W = SEQ // NW
CS = 8
NCHUNK = S_PER_W // CS
NBUF = 3


def _emb_kernel(x_hbm, tgt_hbm, pos_hbm, out_hbm,
                idx_v, pb0, pb1, pb2, rb0, rb1, rb2,
                isem, p0, p1, p2, g0, g1, g2, t0, t1, t2):
    wid = lax.axis_index("s") * NC + lax.axis_index("c")
    base = wid * S_PER_W

    pbufs = (pb0, pb1, pb2)
    rbufs = (rb0, rb1, rb2)
    gsems = (g0, g1, g2)
    psems = (p0, p1, p2)
    ssems = (t0, t1, t2)

    def fire_gather_b(c, b):
        rb = rbufs[c % NBUF]
        return pltpu.async_copy(tgt_hbm.at[idx_v.at[b, pl.ds(c * CS, CS)]],
                                rb.at[pl.ds(b * CS, CS)], gsems[c % NBUF])

    def fire_gather(c):
        return [fire_gather_b(c, b) for b in range(BATCH)]

    def fire_pos(c):
        return pltpu.async_copy(pos_hbm.at[pl.ds(base + c * CS, CS)],
                                pbufs[c % NBUF], psems[c % NBUF])

    def fire_scatter(c):
        rb = rbufs[c % NBUF]
        return [
            pltpu.async_copy(rb.at[pl.ds(b * CS, CS)],
                             out_hbm.at[b, pl.ds(base + c * CS, CS)],
                             ssems[c % NBUF])
            for b in range(BATCH)
        ]

    gh = [None] * NCHUNK
    ph = [None] * NCHUNK
    sh = [None] * NCHUNK
    ph[0] = fire_pos(0)
    ph[1] = fire_pos(1)
    idx_cps = [
        pltpu.async_copy(x_hbm.at[b, pl.ds(base, S_PER_W)], idx_v.at[b], isem)
        for b in range(BATCH)
    ]
    for cp in idx_cps:
        cp.wait()
    gh[0] = fire_gather(0)
    gh[1] = fire_gather(1)
    for c in range(NCHUNK):
        for h_ in gh[c]:
            h_.wait()
        ph[c].wait()

        pb = pbufs[c % NBUF]
        rb = rbufs[c % NBUF]

        @plsc.parallel_loop(0, CS, 1)
        def _add(r, pb=pb, rb=rb):
            for k in range(VECS):
                v = pb[r, pl.ds(k * LANES, LANES)]
                for b in range(BATCH):
                    plsc.addupdate(rb.at[b * CS + r, pl.ds(k * LANES, LANES)], v)

        sh[c] = fire_scatter(c)
        if c + 2 < NCHUNK:
            if c >= 1:
                for h_ in sh[c - 1]:
                    h_.wait()
            gh[c + 2] = fire_gather(c + 2)
            ph[c + 2] = fire_pos(c + 2)

    for c in (NCHUNK - 3, NCHUNK - 2, NCHUNK - 1):
        for h_ in sh[c]:
            h_.wait()


def _emb_impl(x, tgt_emb, pos_emb):
    mesh = plsc.VectorSubcoreMesh(core_axis_name="c", subcore_axis_name="s")
    f = functools.partial(
        pl.kernel,
        out_type=jax.ShapeDtypeStruct((BATCH, SEQ, D), jnp.float32),
        mesh=mesh,
        compiler_params=pltpu.CompilerParams(
            disable_bounds_checks=True,
            disable_semaphore_checks=True,
        ),
        scratch_types=[
            pltpu.VMEM((BATCH, S_PER_W), jnp.int32),
            pltpu.VMEM((CS, D), jnp.float32),
            pltpu.VMEM((CS, D), jnp.float32),
            pltpu.VMEM((CS, D), jnp.float32),
            pltpu.VMEM((BATCH * CS, D), jnp.float32),
            pltpu.VMEM((BATCH * CS, D), jnp.float32),
            pltpu.VMEM((BATCH * CS, D), jnp.float32),
            pltpu.SemaphoreType.DMA,
            pltpu.SemaphoreType.DMA,
            pltpu.SemaphoreType.DMA,
            pltpu.SemaphoreType.DMA,
            pltpu.SemaphoreType.DMA,
            pltpu.SemaphoreType.DMA,
            pltpu.SemaphoreType.DMA,
            pltpu.SemaphoreType.DMA,
            pltpu.SemaphoreType.DMA,
            pltpu.SemaphoreType.DMA,
        ],
    )(_emb_kernel)
    return f(x, tgt_emb, pos_emb)


_emb = jax.jit(_emb_impl)


def kernel(x, tgt_emb, pos_emb):
    return _emb(x.astype(jnp.int32), tgt_emb, pos_emb)


# speedup vs baseline: 1.0422x; 1.0422x over previous
import functools

import jax
import jax.numpy as jnp
from jax import lax
from jax.experimental import pallas as pl
from jax.experimental.pallas import tpu as pltpu
from jax.experimental.pallas import tpu_sc as plsc

D = 768
BATCH = 4
SEQ = 2048
LANES = 16
VECS = D // LANES

_info = plsc.get_sparse_core_info()
NC = _info.num_cores
NS = _info.num_subcores
NW = NC * NS
S_PER_W = SEQ // NW
CS = 16
NCHUNK = S_PER_W // CS
NBUF = 2
PB_OFF = 0
RB_OFF = NBUF * CS
ROWS = NBUF * CS + NBUF * BATCH * CS


def _emb_kernel(x_hbm, tgt_hbm, pos_hbm, out_hbm,
                idx_v, buf, g0, g1, t0, t1):
    wid = lax.axis_index("s") * NC + lax.axis_index("c")
    base = wid * S_PER_W

    gsems = (g0, g1)
    ssems = (t0, t1)

    def pb_rows(c):
        return PB_OFF + (c % NBUF) * CS

    def rb_rows(c):
        return RB_OFF + (c % NBUF) * BATCH * CS

    def fire_gather(c):
        r0 = rb_rows(c)
        return [
            pltpu.async_copy(tgt_hbm.at[idx_v.at[b, pl.ds(c * CS, CS)]],
                             buf.at[pl.ds(r0 + b * CS, CS)], gsems[c % NBUF])
            for b in range(BATCH)
        ]

    def fire_pos(c):
        return pltpu.async_copy(pos_hbm.at[pl.ds(base + c * CS, CS)],
                                buf.at[pl.ds(pb_rows(c), CS)], gsems[c % NBUF])

    def fire_scatter(c):
        r0 = rb_rows(c)
        return [
            pltpu.async_copy(buf.at[pl.ds(r0 + b * CS, CS)],
                             out_hbm.at[b, pl.ds(base + c * CS, CS)],
                             ssems[c % NBUF])
            for b in range(BATCH)
        ]

    gh = [None] * NCHUNK
    ph = [None] * NCHUNK
    sh = [None] * NCHUNK
    ph[0] = fire_pos(0)
    ph[1] = fire_pos(1)
    idx_cps = [
        pltpu.async_copy(x_hbm.at[b, pl.ds(base, S_PER_W)], idx_v.at[b], t0)
        for b in range(BATCH)
    ]
    for cp in idx_cps:
        cp.wait()
    gh[0] = fire_gather(0)
    gh[1] = fire_gather(1)
    for c in range(NCHUNK):
        for h_ in gh[c]:
            h_.wait()
        ph[c].wait()

        p0 = pb_rows(c)
        r0 = rb_rows(c)

        @plsc.parallel_loop(0, CS, 1)
        def _add(r, p0=p0, r0=r0):
            for k in range(VECS):
                v = buf[p0 + r, pl.ds(k * LANES, LANES)]
                for b in range(BATCH):
                    plsc.addupdate(
                        buf.at[r0 + b * CS + r, pl.ds(k * LANES, LANES)], v)

        sh[c] = fire_scatter(c)
        if c + 2 < NCHUNK:
            for h_ in sh[c]:
                h_.wait()
            gh[c + 2] = fire_gather(c + 2)
            ph[c + 2] = fire_pos(c + 2)

    for c in (NCHUNK - 2, NCHUNK - 1):
        for h_ in sh[c]:
            h_.wait()


def _emb_impl(x, tgt_emb, pos_emb):
    mesh = plsc.VectorSubcoreMesh(core_axis_name="c", subcore_axis_name="s")
    f = functools.partial(
        pl.kernel,
        out_type=jax.ShapeDtypeStruct((BATCH, SEQ, D), jnp.float32),
        mesh=mesh,
        compiler_params=pltpu.CompilerParams(
            disable_bounds_checks=True,
            disable_semaphore_checks=True,
        ),
        scratch_types=[
            pltpu.VMEM((BATCH, S_PER_W), jnp.int32),
            pltpu.VMEM((ROWS, D), jnp.float32),
            pltpu.SemaphoreType.DMA,
            pltpu.SemaphoreType.DMA,
            pltpu.SemaphoreType.DMA,
            pltpu.SemaphoreType.DMA,
        ],
    )(_emb_kernel)
    return f(x, tgt_emb, pos_emb)


_emb = jax.jit(_emb_impl)


def kernel(x, tgt_emb, pos_emb):
    return _emb(x.astype(jnp.int32), tgt_emb, pos_emb)


# speedup vs baseline: 1.0505x; 1.0080x over previous
import functools

import jax
import jax.numpy as jnp
from jax import lax
from jax.experimental import pallas as pl
from jax.experimental.pallas import tpu as pltpu
from jax.experimental.pallas import tpu_sc as plsc

D = 768
BATCH = 4
SEQ = 2048
LANES = 16
VECS = D // LANES

_info = plsc.get_sparse_core_info()
NC = _info.num_cores
NS = _info.num_subcores
NW = NC * NS
S_PER_W = SEQ // NW
CS = 16
CSIZES = (8, 16, 16, 16, 8)
CSTARTS = (0, 8, 24, 40, 56)
NCHUNK = len(CSIZES)
NBUF = 2
PB_OFF = 0
RB_OFF = NBUF * CS
ROWS = NBUF * CS + NBUF * BATCH * CS


def _emb_kernel(x_hbm, tgt_hbm, pos_hbm, out_hbm,
                idx_v, buf, g0, g1, t0, t1):
    wid = lax.axis_index("s") * NC + lax.axis_index("c")
    base = wid * S_PER_W

    gsems = (g0, g1)
    ssems = (t0, t1)

    def pb_rows(c):
        return PB_OFF + (c % NBUF) * CS

    def rb_rows(c):
        return RB_OFF + (c % NBUF) * BATCH * CS

    def fire_gather(c):
        r0, s0, sz = rb_rows(c), CSTARTS[c], CSIZES[c]
        return [
            pltpu.async_copy(tgt_hbm.at[idx_v.at[b, pl.ds(s0, sz)]],
                             buf.at[pl.ds(r0 + b * CS, sz)], gsems[c % NBUF])
            for b in range(BATCH)
        ]

    def fire_pos(c):
        s0, sz = CSTARTS[c], CSIZES[c]
        return pltpu.async_copy(pos_hbm.at[pl.ds(base + s0, sz)],
                                buf.at[pl.ds(pb_rows(c), sz)], gsems[c % NBUF])

    def fire_scatter(c):
        r0, s0, sz = rb_rows(c), CSTARTS[c], CSIZES[c]
        return [
            pltpu.async_copy(buf.at[pl.ds(r0 + b * CS, sz)],
                             out_hbm.at[b, pl.ds(base + s0, sz)],
                             ssems[c % NBUF])
            for b in range(BATCH)
        ]

    gh = [None] * NCHUNK
    ph = [None] * NCHUNK
    sh = [None] * NCHUNK
    ph[0] = fire_pos(0)
    ph[1] = fire_pos(1)
    idx_cps = [
        pltpu.async_copy(x_hbm.at[b, pl.ds(base, S_PER_W)], idx_v.at[b], t0)
        for b in range(BATCH)
    ]
    for cp in idx_cps:
        cp.wait()
    gh[0] = fire_gather(0)
    gh[1] = fire_gather(1)
    for c in range(NCHUNK):
        for h_ in gh[c]:
            h_.wait()
        ph[c].wait()

        p0 = pb_rows(c)
        r0 = rb_rows(c)

        @plsc.parallel_loop(0, CSIZES[c], 1)
        def _add(r, p0=p0, r0=r0):
            for k in range(VECS):
                v = buf[p0 + r, pl.ds(k * LANES, LANES)]
                for b in range(BATCH):
                    plsc.addupdate(
                        buf.at[r0 + b * CS + r, pl.ds(k * LANES, LANES)], v)

        sh[c] = fire_scatter(c)
        if c + 2 < NCHUNK:
            for h_ in sh[c]:
                h_.wait()
            gh[c + 2] = fire_gather(c + 2)
            ph[c + 2] = fire_pos(c + 2)

    for c in (NCHUNK - 2, NCHUNK - 1):
        for h_ in sh[c]:
            h_.wait()


def _emb_impl(x, tgt_emb, pos_emb):
    mesh = plsc.VectorSubcoreMesh(core_axis_name="c", subcore_axis_name="s")
    f = functools.partial(
        pl.kernel,
        out_type=jax.ShapeDtypeStruct((BATCH, SEQ, D), jnp.float32),
        mesh=mesh,
        compiler_params=pltpu.CompilerParams(
            disable_bounds_checks=True,
            disable_semaphore_checks=True,
        ),
        scratch_types=[
            pltpu.VMEM((BATCH, S_PER_W), jnp.int32),
            pltpu.VMEM((ROWS, D), jnp.float32),
            pltpu.SemaphoreType.DMA,
            pltpu.SemaphoreType.DMA,
            pltpu.SemaphoreType.DMA,
            pltpu.SemaphoreType.DMA,
        ],
    )(_emb_kernel)
    return f(x, tgt_emb, pos_emb)


_emb = jax.jit(_emb_impl)


def kernel(x, tgt_emb, pos_emb):
    return _emb(x.astype(jnp.int32), tgt_emb, pos_emb)
